# baseline (device time: 322839 ns/iter reference)
import jax
import jax.numpy as jnp
from jax import lax
from jax.experimental import pallas as pl
from jax.experimental.pallas import tpu as pltpu

VB = 512
CHUNK = 256


def kernel(ids, E):
    T = ids.shape[0]
    V_local, D = E.shape
    HALF = T // 2
    NB = V_local // VB
    NC = HALF // CHUNK

    ids2 = ids.reshape(T, 1).astype(jnp.int32)

    def body(ids_ref, e_ref, out_ref,
             e_stage, acc, z_rbuf, red, x_rbuf,
             e_sem, st_sems, st2_sem,
             z_send, z_recv, x_send, x_recv,
             z_credit, x_credit):
        my_x = lax.axis_index("x")
        my_y = lax.axis_index("y")
        my_z = lax.axis_index("z")
        peer_z = (my_x, my_y, 1 - my_z)
        peer_x = (1 - my_x, my_y, my_z)

        barrier = pltpu.get_barrier_semaphore()
        for nbr in (peer_z, peer_x):
            pl.semaphore_signal(barrier, inc=1, device_id=nbr,
                                device_id_type=pl.DeviceIdType.MESH)
        pl.semaphore_wait(barrier, 2)

        my_base = my_x * HALF
        voff = my_z * V_local
        col = lax.broadcasted_iota(jnp.int32, (1, VB), 1)
        GROUP = 2 * CHUNK
        NG = HALF // GROUP

        def e_load(b, slot):
            return pltpu.make_async_copy(
                e_ref.at[pl.ds(b * VB, VB), :], e_stage.at[slot], e_sem.at[slot])

        def gather(g):
            rows = pl.ds(g * GROUP, GROUP)
            idg = ids_ref[pl.ds(my_base + g * GROUP, GROUP), :]
            e_load(0, 0).start()
            for b in range(NB):
                slot = b % 2
                e_load(b, slot).wait()
                if b + 1 < NB:
                    e_load(b + 1, (b + 1) % 2).start()
                oh = (idg == (voff + b * VB + col)).astype(jnp.float32)
                mm = jax.lax.dot_general(
                    oh, e_stage[slot],
                    (((1,), (0,)), ((), ())),
                    preferred_element_type=jnp.float32)
                if b == 0:
                    acc[rows, :] = mm
                else:
                    acc[rows, :] = acc[rows, :] + mm

        gather(0)

        def z_desc(c):
            slot = c % 3
            return pltpu.make_async_remote_copy(
                src_ref=acc.at[pl.ds(c * CHUNK, CHUNK), :],
                dst_ref=z_rbuf.at[slot],
                send_sem=z_send.at[slot], recv_sem=z_recv.at[slot],
                device_id=peer_z, device_id_type=pl.DeviceIdType.MESH)

        def st_desc(c):
            return pltpu.make_async_copy(
                red.at[c % 2],
                out_ref.at[pl.ds(my_base + c * CHUNK, CHUNK), :],
                st_sems.at[c % 2])

        def x_desc(c):
            slot = c % 2
            return pltpu.make_async_remote_copy(
                src_ref=red.at[slot],
                dst_ref=x_rbuf.at[slot],
                send_sem=x_send.at[slot], recv_sem=x_recv.at[slot],
                device_id=peer_x, device_id_type=pl.DeviceIdType.MESH)

        z_desc(0).start()
        z_desc(1).start()
        for c in range(NC):
            slot = c % 2
            if c % 2 == 0 and c // 2 + 1 < NG:
                gather(c // 2 + 1)
            if c + 2 < NC:
                if c >= 1:
                    z_desc(c + 2).wait_send()
                    pl.semaphore_wait(z_credit, 1)
                z_desc(c + 2).start()
            z_desc(c).wait_recv()
            if c >= 2:
                x_desc(c).wait_send()
                st_desc(c - 2).wait()
            red[slot] = acc[pl.ds(c * CHUNK, CHUNK), :] + z_rbuf[c % 3]
            if c < NC - 3:
                pl.semaphore_signal(z_credit, inc=1, device_id=peer_z,
                                    device_id_type=pl.DeviceIdType.MESH)
            if c >= 2:
                pl.semaphore_wait(x_credit, 1)
            x_desc(c).start()
            st_desc(c).start()
            if c >= 1:
                x_desc(c - 1).wait_recv()
                st2 = pltpu.make_async_copy(
                    x_rbuf.at[(c - 1) % 2],
                    out_ref.at[
                        pl.ds((1 - my_x) * HALF + (c - 1) * CHUNK, CHUNK), :],
                    st2_sem)
                st2.start()
                st2.wait()
                if c - 1 < NC - 2:
                    pl.semaphore_signal(x_credit, inc=1, device_id=peer_x,
                                        device_id_type=pl.DeviceIdType.MESH)

        x_desc(NC - 1).wait_recv()
        st2 = pltpu.make_async_copy(
            x_rbuf.at[(NC - 1) % 2],
            out_ref.at[pl.ds((1 - my_x) * HALF + (NC - 1) * CHUNK, CHUNK), :],
            st2_sem)
        st2.start()
        st2.wait()
        for c in range(NC - 3, NC):
            z_desc(c).wait_send()
        x_desc(NC - 2).wait_send()
        x_desc(NC - 1).wait_send()
        st_desc(NC - 2).wait()
        st_desc(NC - 1).wait()

    out = pl.pallas_call(
        body,
        out_shape=jax.ShapeDtypeStruct((T, D), jnp.float32),
        in_specs=[
            pl.BlockSpec(memory_space=pltpu.MemorySpace.VMEM),
            pl.BlockSpec(memory_space=pl.ANY),
        ],
        out_specs=pl.BlockSpec(memory_space=pl.ANY),
        scratch_shapes=[
            pltpu.VMEM((2, VB, D), jnp.float32),
            pltpu.VMEM((HALF, D), jnp.float32),
            pltpu.VMEM((3, CHUNK, D), jnp.float32),
            pltpu.VMEM((2, CHUNK, D), jnp.float32),
            pltpu.VMEM((2, CHUNK, D), jnp.float32),
            pltpu.SemaphoreType.DMA((2,)),
            pltpu.SemaphoreType.DMA((2,)),
            pltpu.SemaphoreType.DMA,
            pltpu.SemaphoreType.DMA((3,)),
            pltpu.SemaphoreType.DMA((3,)),
            pltpu.SemaphoreType.DMA((2,)),
            pltpu.SemaphoreType.DMA((2,)),
            pltpu.SemaphoreType.REGULAR,
            pltpu.SemaphoreType.REGULAR,
        ],
        compiler_params=pltpu.CompilerParams(
            collective_id=0, vmem_limit_bytes=100 * 1024 * 1024),
    )(ids2, E)
    return out


# device time: 314603 ns/iter; 1.0262x vs baseline; 1.0262x over previous
import jax
import jax.numpy as jnp
from jax import lax
from jax.experimental import pallas as pl
from jax.experimental.pallas import tpu as pltpu

VB = 512
CHUNK = 256


def kernel(ids, E):
    T = ids.shape[0]
    V_local, D = E.shape
    HALF = T // 2
    NB = V_local // VB
    NC = HALF // CHUNK

    ids2 = ids.reshape(T, 1).astype(jnp.int32)

    def body(ids_ref, e_ref, out_ref,
             e_stage, acc, z_rbuf, red, x_rbuf,
             e_sem, st_sems, st2_sem,
             z_send, z_recv, x_send, x_recv,
             z_credit, x_credit):
        my_x = lax.axis_index("x")
        my_y = lax.axis_index("y")
        my_z = lax.axis_index("z")
        peer_z = (my_x, my_y, 1 - my_z)
        peer_x = (1 - my_x, my_y, my_z)

        barrier = pltpu.get_barrier_semaphore()
        for nbr in (peer_z, peer_x):
            pl.semaphore_signal(barrier, inc=1, device_id=nbr,
                                device_id_type=pl.DeviceIdType.MESH)
        pl.semaphore_wait(barrier, 2)

        my_base = my_x * HALF
        voff = my_z * V_local
        col = lax.broadcasted_iota(jnp.int32, (1, VB), 1)
        ids_my = ids_ref[pl.ds(my_base, HALF), :]

        def e_load(b, slot):
            return pltpu.make_async_copy(
                e_ref.at[pl.ds(b * VB, VB), :], e_stage.at[slot], e_sem.at[slot])

        e_load(0, 0).start()
        for b in range(NB):
            eslot = b % 2
            e_load(b, eslot).wait()
            if b + 1 < NB:
                e_load(b + 1, (b + 1) % 2).start()
            oh = (ids_my == (voff + b * VB + col)).astype(jnp.float32)
            mm = jax.lax.dot_general(
                oh, e_stage[eslot],
                (((1,), (0,)), ((), ())),
                preferred_element_type=jnp.float32)
            if b == 0:
                acc[...] = mm
            else:
                acc[...] = acc[...] + mm

        def z_desc(c):
            slot = c % 3
            return pltpu.make_async_remote_copy(
                src_ref=acc.at[pl.ds(c * CHUNK, CHUNK), :],
                dst_ref=z_rbuf.at[slot],
                send_sem=z_send.at[slot], recv_sem=z_recv.at[slot],
                device_id=peer_z, device_id_type=pl.DeviceIdType.MESH)

        def st_desc(c):
            return pltpu.make_async_copy(
                red.at[c % 2],
                out_ref.at[pl.ds(my_base + c * CHUNK, CHUNK), :],
                st_sems.at[c % 2])

        def x_desc(c):
            slot = c % 2
            return pltpu.make_async_remote_copy(
                src_ref=red.at[slot],
                dst_ref=x_rbuf.at[slot],
                send_sem=x_send.at[slot], recv_sem=x_recv.at[slot],
                device_id=peer_x, device_id_type=pl.DeviceIdType.MESH)

        z_desc(0).start()
        z_desc(1).start()
        for c in range(NC):
            slot = c % 2
            if c + 2 < NC:
                if c >= 1:
                    z_desc(c + 2).wait_send()
                    pl.semaphore_wait(z_credit, 1)
                z_desc(c + 2).start()
            z_desc(c).wait_recv()
            if c >= 2:
                x_desc(c).wait_send()
                st_desc(c - 2).wait()
            red[slot] = acc[pl.ds(c * CHUNK, CHUNK), :] + z_rbuf[c % 3]
            if c < NC - 3:
                pl.semaphore_signal(z_credit, inc=1, device_id=peer_z,
                                    device_id_type=pl.DeviceIdType.MESH)
            if c >= 2:
                pl.semaphore_wait(x_credit, 1)
            x_desc(c).start()
            st_desc(c).start()
            if c >= 1:
                x_desc(c - 1).wait_recv()
                st2 = pltpu.make_async_copy(
                    x_rbuf.at[(c - 1) % 2],
                    out_ref.at[
                        pl.ds((1 - my_x) * HALF + (c - 1) * CHUNK, CHUNK), :],
                    st2_sem)
                st2.start()
                st2.wait()
                if c - 1 < NC - 2:
                    pl.semaphore_signal(x_credit, inc=1, device_id=peer_x,
                                        device_id_type=pl.DeviceIdType.MESH)

        x_desc(NC - 1).wait_recv()
        st2 = pltpu.make_async_copy(
            x_rbuf.at[(NC - 1) % 2],
            out_ref.at[pl.ds((1 - my_x) * HALF + (NC - 1) * CHUNK, CHUNK), :],
            st2_sem)
        st2.start()
        st2.wait()
        for c in range(NC - 3, NC):
            z_desc(c).wait_send()
        x_desc(NC - 2).wait_send()
        x_desc(NC - 1).wait_send()
        st_desc(NC - 2).wait()
        st_desc(NC - 1).wait()

    out = pl.pallas_call(
        body,
        out_shape=jax.ShapeDtypeStruct((T, D), jnp.float32),
        in_specs=[
            pl.BlockSpec(memory_space=pltpu.MemorySpace.VMEM),
            pl.BlockSpec(memory_space=pl.ANY),
        ],
        out_specs=pl.BlockSpec(memory_space=pl.ANY),
        scratch_shapes=[
            pltpu.VMEM((2, VB, D), jnp.float32),
            pltpu.VMEM((HALF, D), jnp.float32),
            pltpu.VMEM((3, CHUNK, D), jnp.float32),
            pltpu.VMEM((2, CHUNK, D), jnp.float32),
            pltpu.VMEM((2, CHUNK, D), jnp.float32),
            pltpu.SemaphoreType.DMA((2,)),
            pltpu.SemaphoreType.DMA((2,)),
            pltpu.SemaphoreType.DMA,
            pltpu.SemaphoreType.DMA((3,)),
            pltpu.SemaphoreType.DMA((3,)),
            pltpu.SemaphoreType.DMA((2,)),
            pltpu.SemaphoreType.DMA((2,)),
            pltpu.SemaphoreType.REGULAR,
            pltpu.SemaphoreType.REGULAR,
        ],
        compiler_params=pltpu.CompilerParams(
            collective_id=0, vmem_limit_bytes=100 * 1024 * 1024),
    )(ids2, E)
    return out


# device time: 303881 ns/iter; 1.0624x vs baseline; 1.0353x over previous
import jax
import jax.numpy as jnp
from jax import lax
from jax.experimental import pallas as pl
from jax.experimental.pallas import tpu as pltpu

VB = 512
CHUNK = 256


def kernel(ids, E):
    T = ids.shape[0]
    V_local, D = E.shape
    HALF = T // 2
    NB = V_local // VB
    NC = HALF // CHUNK

    ids2 = ids.reshape(T, 1).astype(jnp.int32)

    def body(ids_ref, e_ref, out_ref,
             e_stage, acc, z_rbuf, red, x_rbuf,
             e_sem, st_sems, st2_sem,
             z_send, z_recv, x_send, x_recv,
             z_credit, x_credit):
        my_x = lax.axis_index("x")
        my_y = lax.axis_index("y")
        my_z = lax.axis_index("z")
        peer_z = (my_x, my_y, 1 - my_z)
        peer_x = (1 - my_x, my_y, my_z)

        barrier = pltpu.get_barrier_semaphore()
        for nbr in (peer_z, peer_x):
            pl.semaphore_signal(barrier, inc=1, device_id=nbr,
                                device_id_type=pl.DeviceIdType.MESH)
        pl.semaphore_wait(barrier, 2)

        my_base = my_x * HALF
        voff = my_z * V_local
        col = lax.broadcasted_iota(jnp.int32, (1, VB), 1)
        ids_my = ids_ref[pl.ds(my_base, HALF), :]

        def e_load(b, slot):
            return pltpu.make_async_copy(
                e_ref.at[pl.ds(b * VB, VB), :], e_stage.at[slot], e_sem.at[slot])

        e_load(0, 0).start()
        for b in range(NB):
            eslot = b % 2
            e_load(b, eslot).wait()
            if b + 1 < NB:
                e_load(b + 1, (b + 1) % 2).start()
            oh = (ids_my == (voff + b * VB + col)).astype(jnp.float32)
            mm = jax.lax.dot_general(
                oh, e_stage[eslot],
                (((1,), (0,)), ((), ())),
                preferred_element_type=jnp.float32)
            if b == 0:
                acc[...] = mm
            else:
                acc[...] = acc[...] + mm

        def z_desc(c):
            slot = c % 3
            return pltpu.make_async_remote_copy(
                src_ref=acc.at[pl.ds(c * CHUNK, CHUNK), :],
                dst_ref=z_rbuf.at[slot],
                send_sem=z_send.at[slot], recv_sem=z_recv.at[slot],
                device_id=peer_z, device_id_type=pl.DeviceIdType.MESH)

        def x_desc(c):
            slot = c % 2
            return pltpu.make_async_remote_copy(
                src_ref=red.at[slot],
                dst_ref=x_rbuf.at[slot],
                send_sem=x_send.at[slot], recv_sem=x_recv.at[slot],
                device_id=peer_x, device_id_type=pl.DeviceIdType.MESH)

        def st_desc(c):
            return pltpu.make_async_copy(
                red.at[c % 2],
                out_ref.at[pl.ds(my_base + c * CHUNK, CHUNK), :],
                st_sems.at[c % 2])

        z_desc(0).start()
        z_desc(1).start()
        for c in range(NC):
            slot = c % 2
            if c + 2 < NC:
                if c >= 1:
                    z_desc(c + 2).wait_send()
                    pl.semaphore_wait(z_credit, 1)
                z_desc(c + 2).start()
            z_desc(c).wait_recv()
            if c >= 2:
                x_desc(c).wait_send()
                st_desc(c - 2).wait()
            red[slot] = acc[pl.ds(c * CHUNK, CHUNK), :] + z_rbuf[c % 3]
            if c < NC - 3:
                pl.semaphore_signal(z_credit, inc=1, device_id=peer_z,
                                    device_id_type=pl.DeviceIdType.MESH)
            if c >= 2:
                pl.semaphore_wait(x_credit, 1)
            x_desc(c).start()
            st_desc(c).start()
            if c >= 1:
                x_desc(c - 1).wait_recv()
                st2 = pltpu.make_async_copy(
                    x_rbuf.at[(c - 1) % 2],
                    out_ref.at[
                        pl.ds((1 - my_x) * HALF + (c - 1) * CHUNK, CHUNK), :],
                    st2_sem)
                st2.start()
                st2.wait()
                if c - 1 < NC - 2:
                    pl.semaphore_signal(x_credit, inc=1, device_id=peer_x,
                                        device_id_type=pl.DeviceIdType.MESH)

        x_desc(NC - 1).wait_recv()
        st2 = pltpu.make_async_copy(
            x_rbuf.at[(NC - 1) % 2],
            out_ref.at[pl.ds((1 - my_x) * HALF + (NC - 1) * CHUNK, CHUNK), :],
            st2_sem)
        st2.start()
        st2.wait()
        for c in range(NC - 3, NC):
            z_desc(c).wait_send()
        x_desc(NC - 2).wait_send()
        x_desc(NC - 1).wait_send()
        st_desc(NC - 2).wait()
        st_desc(NC - 1).wait()

    out = pl.pallas_call(
        body,
        out_shape=jax.ShapeDtypeStruct((T, D), jnp.float32),
        in_specs=[
            pl.BlockSpec(memory_space=pltpu.MemorySpace.VMEM),
            pl.BlockSpec(memory_space=pl.ANY),
        ],
        out_specs=pl.BlockSpec(memory_space=pl.ANY),
        scratch_shapes=[
            pltpu.VMEM((2, VB, D), jnp.float32),
            pltpu.VMEM((HALF, D), jnp.float32),
            pltpu.VMEM((3, CHUNK, D), jnp.float32),
            pltpu.VMEM((2, CHUNK, D), jnp.float32),
            pltpu.VMEM((2, CHUNK, D), jnp.float32),
            pltpu.SemaphoreType.DMA((2,)),
            pltpu.SemaphoreType.DMA((2,)),
            pltpu.SemaphoreType.DMA,
            pltpu.SemaphoreType.DMA((3,)),
            pltpu.SemaphoreType.DMA((3,)),
            pltpu.SemaphoreType.DMA((2,)),
            pltpu.SemaphoreType.DMA((2,)),
            pltpu.SemaphoreType.REGULAR,
            pltpu.SemaphoreType.REGULAR,
        ],
        compiler_params=pltpu.CompilerParams(
            collective_id=0, vmem_limit_bytes=100 * 1024 * 1024),
    )(ids2, E)
    return out
